# baseline (speedup 1.0000x reference)
import functools
import math

import jax
import jax.numpy as jnp
from jax import lax
from jax.experimental import pallas as pl
from jax.experimental.pallas import tpu as pltpu
from jax.experimental.pallas import tpu_sc as plsc

D_MODEL = 768
LANES = 16
NUM_CORES = 2
NUM_SUBCORES = 16
NUM_WORKERS = NUM_CORES * NUM_SUBCORES
CHUNK = 128
SCALE = 1.0 / math.sqrt(D_MODEL)


def _sc_body(word_hbm, task_hbm, seg_hbm, tid_hbm, sid_hbm, ids_hbm, out_hbm,
             small_v, comb_v, ids_v, tid_v, sid_v, buf, sem):
    n_tok = ids_hbm.shape[0]
    tok_per_w = n_tok // NUM_WORKERS
    wid = lax.axis_index("s") * NUM_CORES + lax.axis_index("c")
    base = wid * tok_per_w

    pltpu.sync_copy(task_hbm, small_v.at[pl.ds(0, 3)])
    pltpu.sync_copy(seg_hbm, small_v.at[pl.ds(3, 3)])

    @pl.loop(0, D_MODEL // LANES)
    def _build(j):
        col = j * LANES
        for t in range(3):
            tv = small_v[t, pl.ds(col, LANES)]
            for s in range(3):
                sv = small_v[3 + s, pl.ds(col, LANES)]
                comb_v[t * 3 + s, pl.ds(col, LANES)] = tv + sv * SCALE

    for ch in range(tok_per_w // CHUNK):
        off = base + ch * CHUNK
        pltpu.sync_copy(ids_hbm.at[pl.ds(off, CHUNK)], ids_v)
        pltpu.sync_copy(tid_hbm.at[pl.ds(off, CHUNK)], tid_v)
        pltpu.sync_copy(sid_hbm.at[pl.ds(off, CHUNK)], sid_v)
        pltpu.async_copy(word_hbm.at[ids_v], buf, sem).wait()
        for g in range(CHUNK // LANES):
            tvec = tid_v[pl.ds(g * LANES, LANES)]
            svec = sid_v[pl.ds(g * LANES, LANES)]
            cvec = tvec * 3 + svec
            rows = lax.iota(jnp.int32, LANES) + (g * LANES)

            @pl.loop(0, D_MODEL, unroll=16)
            def _add(j, cvec=cvec, rows=rows):
                jv = jnp.full((LANES,), 0, jnp.int32) + j
                vals = plsc.load_gather(comb_v, [cvec, jv])
                plsc.addupdate_scatter(buf, [rows, jv], vals)

        pltpu.sync_copy(buf, out_hbm.at[pl.ds(off, CHUNK)])


@functools.lru_cache(maxsize=None)
def _make_sc_call(n_tok: int):
    return pl.kernel(
        _sc_body,
        out_type=jax.ShapeDtypeStruct((n_tok, D_MODEL), jnp.float32),
        mesh=plsc.VectorSubcoreMesh(core_axis_name="c", subcore_axis_name="s"),
        compiler_params=pltpu.CompilerParams(
            use_tc_tiling_on_sc=False, needs_layout_passes=False),
        scratch_types=[
            pltpu.VMEM((6, D_MODEL), jnp.float32),
            pltpu.VMEM((9, D_MODEL), jnp.float32),
            pltpu.VMEM((CHUNK,), jnp.int32),
            pltpu.VMEM((CHUNK,), jnp.int32),
            pltpu.VMEM((CHUNK,), jnp.int32),
            pltpu.VMEM((CHUNK, D_MODEL), jnp.float32),
            pltpu.SemaphoreType.DMA,
        ],
    )


@jax.jit
def kernel(input_ids, task_ids, segment_ids, word_table, task_table, segment_table):
    shape = input_ids.shape
    ids = input_ids.reshape(-1).astype(jnp.int32)
    tid = task_ids.reshape(-1).astype(jnp.int32)
    sid = segment_ids.reshape(-1).astype(jnp.int32)
    out = _make_sc_call(ids.shape[0])(
        word_table, task_table, segment_table, tid, sid, ids)
    return out.reshape(shape + (D_MODEL,))


# speedup vs baseline: 2.1828x; 2.1828x over previous
import functools
import math

import jax
import jax.numpy as jnp
from jax import lax
from jax.experimental import pallas as pl
from jax.experimental.pallas import tpu as pltpu
from jax.experimental.pallas import tpu_sc as plsc

D_MODEL = 768
LANES = 16
NUM_CORES = 2
NUM_SUBCORES = 16
NUM_WORKERS = NUM_CORES * NUM_SUBCORES
CHUNK = 128
SCALE = 1.0 / math.sqrt(D_MODEL)


def _sc_body(word_hbm, task_hbm, seg_hbm, tid_hbm, sid_hbm, ids_hbm, out_hbm,
             small_v, comb_v, ids_v, tid_v, sid_v, cidx_v, buf, sem):
    n_tok = ids_hbm.shape[0]
    tok_per_w = n_tok // NUM_WORKERS
    wid = lax.axis_index("s") * NUM_CORES + lax.axis_index("c")
    base = wid * tok_per_w

    pltpu.sync_copy(task_hbm, small_v.at[pl.ds(0, 3)])
    pltpu.sync_copy(seg_hbm, small_v.at[pl.ds(3, 3)])

    @pl.loop(0, D_MODEL // LANES)
    def _build(j):
        col = j * LANES
        for t in range(3):
            tv = small_v[t, pl.ds(col, LANES)]
            for s in range(3):
                sv = small_v[3 + s, pl.ds(col, LANES)]
                comb_v[pl.ds((t * 3 + s) * D_MODEL + col, LANES)] = tv + sv * SCALE

    lane_iota = lax.iota(jnp.int32, LANES)

    for ch in range(tok_per_w // CHUNK):
        off = base + ch * CHUNK
        pltpu.sync_copy(ids_hbm.at[pl.ds(off, CHUNK)], ids_v)
        pltpu.sync_copy(tid_hbm.at[pl.ds(off, CHUNK)], tid_v)
        pltpu.sync_copy(sid_hbm.at[pl.ds(off, CHUNK)], sid_v)
        for g in range(CHUNK // LANES):
            sl = pl.ds(g * LANES, LANES)
            cidx_v[sl] = tid_v[sl] * 3 + sid_v[sl]
        pltpu.async_copy(word_hbm.at[ids_v], buf, sem).wait()

        @pl.loop(0, CHUNK)
        def _add(t, ch=ch):
            ctv = plsc.load_gather(cidx_v, [jnp.full((LANES,), 0, jnp.int32) + t])
            cbase = ctv * D_MODEL + lane_iota
            for j in range(D_MODEL // LANES):
                cvals = plsc.load_gather(comb_v, [cbase + (j * LANES)])
                csl = pl.ds(j * LANES, LANES)
                buf[t, csl] = buf[t, csl] + cvals

        pltpu.sync_copy(buf, out_hbm.at[pl.ds(off, CHUNK)])


@functools.lru_cache(maxsize=None)
def _make_sc_call(n_tok: int):
    return pl.kernel(
        _sc_body,
        out_type=jax.ShapeDtypeStruct((n_tok, D_MODEL), jnp.float32),
        mesh=plsc.VectorSubcoreMesh(core_axis_name="c", subcore_axis_name="s"),
        compiler_params=pltpu.CompilerParams(
            use_tc_tiling_on_sc=False, needs_layout_passes=False),
        scratch_types=[
            pltpu.VMEM((6, D_MODEL), jnp.float32),
            pltpu.VMEM((9 * D_MODEL,), jnp.float32),
            pltpu.VMEM((CHUNK,), jnp.int32),
            pltpu.VMEM((CHUNK,), jnp.int32),
            pltpu.VMEM((CHUNK,), jnp.int32),
            pltpu.VMEM((CHUNK,), jnp.int32),
            pltpu.VMEM((CHUNK, D_MODEL), jnp.float32),
            pltpu.SemaphoreType.DMA,
        ],
    )


@jax.jit
def kernel(input_ids, task_ids, segment_ids, word_table, task_table, segment_table):
    shape = input_ids.shape
    ids = input_ids.reshape(-1).astype(jnp.int32)
    tid = task_ids.reshape(-1).astype(jnp.int32)
    sid = segment_ids.reshape(-1).astype(jnp.int32)
    out = _make_sc_call(ids.shape[0])(
        word_table, task_table, segment_table, tid, sid, ids)
    return out.reshape(shape + (D_MODEL,))
